# Initial kernel scaffold; baseline (speedup 1.0000x reference)
import jax
import jax.numpy as jnp
from jax.experimental import pallas as pl


def kernel(A_assignments, B_assignments, A_codebook, B_codebook):
    raise NotImplementedError("write your pallas kernel here")


# speedup vs baseline: 60.6041x; 60.6041x over previous
import jax
import jax.numpy as jnp
from jax.experimental import pallas as pl
from jax.experimental.pallas import tpu as pltpu

D_OUT = 4096
D_IN = 4096
RANK = 64
N_CODES = 16

BM = 512
BN = 2048


def _dequant(idx, codebook_row):
    out = jnp.full(idx.shape, codebook_row[0, 0], jnp.float32)
    for p in range(1, N_CODES):
        out = jnp.where(idx == p, codebook_row[0, p], out)
    return out


def _fused_kernel(a_idx_ref, b_idx_ref, ca_ref, cb_ref, out_ref):
    a = _dequant(a_idx_ref[...], ca_ref[...])
    b = _dequant(b_idx_ref[...], cb_ref[...])
    out_ref[...] = jax.lax.dot_general(
        a, b, (((1,), (0,)), ((), ())),
        preferred_element_type=jnp.float32,
        precision=jax.lax.Precision.DEFAULT,
    )


def kernel(A_assignments, B_assignments, A_codebook, B_codebook):
    ca = A_codebook.reshape(1, N_CODES).astype(jnp.float32)
    cb = B_codebook.reshape(1, N_CODES).astype(jnp.float32)
    grid = (D_OUT // BM, D_IN // BN)
    return pl.pallas_call(
        _fused_kernel,
        grid=grid,
        in_specs=[
            pl.BlockSpec((BM, RANK), lambda i, j: (i, 0)),
            pl.BlockSpec((RANK, BN), lambda i, j: (0, j)),
            pl.BlockSpec((1, N_CODES), lambda i, j: (0, 0)),
            pl.BlockSpec((1, N_CODES), lambda i, j: (0, 0)),
        ],
        out_specs=pl.BlockSpec((BM, BN), lambda i, j: (i, j)),
        out_shape=jax.ShapeDtypeStruct((D_OUT, D_IN), jnp.float32),
        compiler_params=pltpu.CompilerParams(
            dimension_semantics=("parallel", "parallel"),
        ),
    )(A_assignments, B_assignments, ca, cb)


# speedup vs baseline: 63.9060x; 1.0545x over previous
import jax
import jax.numpy as jnp
from jax.experimental import pallas as pl
from jax.experimental.pallas import tpu as pltpu

D_OUT = 4096
D_IN = 4096
RANK = 64
N_CODES = 16

BM = 512
BN = 2048


def _dequant(idx, codebook_row):
    out = jnp.full(idx.shape, codebook_row[0, 0], jnp.float32)
    for p in range(1, N_CODES):
        out = jnp.where(idx == p, codebook_row[0, p], out)
    return out


def _fused_kernel(a_idx_ref, b_idx_ref, ca_ref, cb_ref, out_ref,
                  a_deq_ref, b_deq_ref):
    i = pl.program_id(0)
    j = pl.program_id(1)

    @pl.when((i == 0) & (j == 0))
    def _():
        a_deq_ref[...] = _dequant(a_idx_ref[...], ca_ref[...])
        b_deq_ref[...] = _dequant(b_idx_ref[...], cb_ref[...])

    a = a_deq_ref[pl.ds(i * BM, BM), :]
    b = b_deq_ref[:, pl.ds(j * BN, BN)]
    out_ref[...] = jax.lax.dot_general(
        a, b, (((1,), (0,)), ((), ())),
        preferred_element_type=jnp.float32,
        precision=jax.lax.Precision.DEFAULT,
    )


def kernel(A_assignments, B_assignments, A_codebook, B_codebook):
    ca = A_codebook.reshape(1, N_CODES).astype(jnp.float32)
    cb = B_codebook.reshape(1, N_CODES).astype(jnp.float32)
    grid = (D_OUT // BM, D_IN // BN)
    return pl.pallas_call(
        _fused_kernel,
        grid=grid,
        in_specs=[
            pl.BlockSpec((D_OUT, RANK), lambda i, j: (0, 0)),
            pl.BlockSpec((RANK, D_IN), lambda i, j: (0, 0)),
            pl.BlockSpec((1, N_CODES), lambda i, j: (0, 0)),
            pl.BlockSpec((1, N_CODES), lambda i, j: (0, 0)),
        ],
        out_specs=pl.BlockSpec((BM, BN), lambda i, j: (i, j)),
        out_shape=jax.ShapeDtypeStruct((D_OUT, D_IN), jnp.float32),
        scratch_shapes=[
            pltpu.VMEM((D_OUT, RANK), jnp.float32),
            pltpu.VMEM((RANK, D_IN), jnp.float32),
        ],
        compiler_params=pltpu.CompilerParams(
            dimension_semantics=("arbitrary", "arbitrary"),
        ),
    )(A_assignments, B_assignments, ca, cb)


# speedup vs baseline: 65.4765x; 1.0246x over previous
import jax
import jax.numpy as jnp
from jax.experimental import pallas as pl
from jax.experimental.pallas import tpu as pltpu

D_OUT = 4096
D_IN = 4096
RANK = 64
N_CODES = 16

BM = 512
BN = 4096


def _dequant(idx, codebook_row):
    out = jnp.full(idx.shape, codebook_row[0, 0], jnp.float32)
    for p in range(1, N_CODES):
        out = jnp.where(idx == p, codebook_row[0, p], out)
    return out


def _fused_kernel(a_idx_ref, b_idx_ref, ca_ref, cb_ref, out_ref,
                  a_deq_ref, b_deq_ref):
    i = pl.program_id(0)
    j = pl.program_id(1)

    @pl.when((i == 0) & (j == 0))
    def _():
        a_deq_ref[...] = _dequant(a_idx_ref[...], ca_ref[...])
        b_deq_ref[...] = _dequant(b_idx_ref[...], cb_ref[...])

    a = a_deq_ref[pl.ds(i * BM, BM), :]
    b = b_deq_ref[:, pl.ds(j * BN, BN)]
    out_ref[...] = jax.lax.dot_general(
        a, b, (((1,), (0,)), ((), ())),
        preferred_element_type=jnp.float32,
        precision=jax.lax.Precision.DEFAULT,
    )


def kernel(A_assignments, B_assignments, A_codebook, B_codebook):
    ca = A_codebook.reshape(1, N_CODES).astype(jnp.float32)
    cb = B_codebook.reshape(1, N_CODES).astype(jnp.float32)
    grid = (D_OUT // BM, D_IN // BN)
    return pl.pallas_call(
        _fused_kernel,
        grid=grid,
        in_specs=[
            pl.BlockSpec((D_OUT, RANK), lambda i, j: (0, 0)),
            pl.BlockSpec((RANK, D_IN), lambda i, j: (0, 0)),
            pl.BlockSpec((1, N_CODES), lambda i, j: (0, 0)),
            pl.BlockSpec((1, N_CODES), lambda i, j: (0, 0)),
        ],
        out_specs=pl.BlockSpec((BM, BN), lambda i, j: (i, j)),
        out_shape=jax.ShapeDtypeStruct((D_OUT, D_IN), jnp.float32),
        scratch_shapes=[
            pltpu.VMEM((D_OUT, RANK), jnp.float32),
            pltpu.VMEM((RANK, D_IN), jnp.float32),
        ],
        compiler_params=pltpu.CompilerParams(
            dimension_semantics=("arbitrary", "arbitrary"),
        ),
    )(A_assignments, B_assignments, ca, cb)
